# baseline (speedup 1.0000x reference)
import functools

import jax
import jax.numpy as jnp
from jax import lax
from jax.experimental import pallas as pl
from jax.experimental.pallas import tpu as pltpu
from jax.experimental.pallas import tpu_sc as plsc

_N = 10000
_E = 320000
_D = 128
_G = 64
_C = 10

_NC = 2
_NS = 16
_NW = _NC * _NS

_EK = 128
_RT = 80
_ERP = _NW * _RT
_EPAD = _ERP * _EK - _E
_NH = _N + 8
_NACC = _N + 8
_NB = 2
_GS = 4
_QR = _EK // _GS
_HALF = 40
_RT0 = 40
_RT1 = 120

_ZR = 632
_ZR_LAST = _N - 15 * _ZR

_ROWBLK = 1000


def _gelu_exact(x):
    return 0.5 * x * (1.0 + lax.erf(x * 0.7071067811865476))


def _zero_acc(zeros_hbm, acc, s):
    @pl.when(s < _NS - 1)
    def _():
        pltpu.sync_copy(zeros_hbm.at[pl.ds(0, _ZR)],
                        acc.at[pl.ds(s * _ZR, _ZR)])

    @pl.when(s == _NS - 1)
    def _():
        pltpu.sync_copy(zeros_hbm.at[pl.ds(0, _ZR_LAST)],
                        acc.at[pl.ds(15 * _ZR, _ZR_LAST)])


def _write_out(acc, out_hbm, c, s):
    @pl.when(s < _NS - 1)
    def _():
        pltpu.sync_copy(acc.at[pl.ds(s * _ZR, _ZR)],
                        out_hbm.at[c, pl.ds(s * _ZR, _ZR)])

    @pl.when(s == _NS - 1)
    def _():
        pltpu.sync_copy(acc.at[pl.ds(15 * _ZR, _ZR_LAST)],
                        out_hbm.at[c, pl.ds(15 * _ZR, _ZR_LAST)])


def _load_slab(src2d, buf, rbase):
    pltpu.sync_copy(src2d.at[pl.ds(rbase, _RT)], buf)


def _sc_degree(dst2d, ones_blk, zeros_blk):
    mesh = plsc.VectorSubcoreMesh(core_axis_name="c", subcore_axis_name="s")

    @functools.partial(
        pl.kernel,
        out_type=jax.ShapeDtypeStruct((_NC, _N, _D), jnp.float32),
        mesh=mesh,
        scratch_types=[
            pltpu.VMEM((_RT, _EK), jnp.int32),
            pltpu.VMEM((_EK, _D), jnp.float32),
            pltpu.VMEM_SHARED((_NACC, _D), jnp.float32),
        ] + [pltpu.SemaphoreType.DMA] * _NB,
    )
    def k(dst_hbm, ones_hbm, zeros_hbm, out_hbm, didxb, ones_v, acc, *sems):
        c = lax.axis_index("c")
        s = lax.axis_index("s")
        t = c * _NS + s
        rbase = t * _RT

        _zero_acc(zeros_hbm, acc, s)
        pltpu.sync_copy(ones_hbm, ones_v)
        _load_slab(dst_hbm, didxb, rbase)
        plsc.subcore_barrier()

        def start(j, b):
            pltpu.async_copy(ones_v, acc.at[didxb.at[j]], sems[b], add=True)

        def wait(j, b):
            pltpu.make_async_copy(ones_v, acc.at[didxb.at[j]], sems[b]).wait()

        for b in range(_NB):
            start(b, b)
        nloops = (_RT - _NB) // _NB

        @pl.loop(0, nloops)
        def _(g):
            for b in range(_NB):
                j = g * _NB + b
                wait(j, b)
                start(j + _NB, b)

        for b in range(_NB):
            j = nloops * _NB + b
            wait(j, b)

        plsc.subcore_barrier()
        _write_out(acc, out_hbm, c, s)

    return k(dst2d, ones_blk, zeros_blk)


def _sc_scatter(hp, src2d, dst2d, zeros_blk):
    mesh = plsc.VectorSubcoreMesh(core_axis_name="c", subcore_axis_name="s")

    @functools.partial(
        pl.kernel,
        out_type=jax.ShapeDtypeStruct((_NC, _N, _D), jnp.float32),
        mesh=mesh,
        scratch_types=[
            pltpu.VMEM((_HALF, _EK), jnp.int32),
            pltpu.VMEM((_HALF, _EK), jnp.int32),
            pltpu.VMEM((_NB, _EK, _D), jnp.float32),
            pltpu.VMEM_SHARED((_NACC, _D), jnp.float32),
        ] + [pltpu.SemaphoreType.DMA] * (_NB * _GS),
    )
    def k(hp_hbm, src_hbm, dst_hbm, zeros_hbm, out_hbm,
          sidxb, didxb, rows, acc, *sems):
        c = lax.axis_index("c")
        s = lax.axis_index("s")

        _zero_acc(zeros_hbm, acc, s)
        plsc.subcore_barrier()

        def start_gather(j, b):
            for q in range(_GS):
                pltpu.async_copy(
                    hp_hbm.at[sidxb.at[j, pl.ds(q * _QR, _QR)]],
                    rows.at[b, pl.ds(q * _QR, _QR)], sems[b * _GS + q])

        def wait_gather(j, b):
            for q in range(_GS):
                pltpu.make_async_copy(
                    hp_hbm.at[sidxb.at[j, pl.ds(q * _QR, _QR)]],
                    rows.at[b, pl.ds(q * _QR, _QR)], sems[b * _GS + q]).wait()

        def scatter(j, b):
            pltpu.sync_copy(rows.at[b], acc.at[didxb.at[j]], add=True)

        nloops = (_HALF - _NB) // _NB

        def pipe(base, nstages):
            for st in range(nstages):
                rbase = base + st * _HALF
                pltpu.sync_copy(src_hbm.at[pl.ds(rbase, _HALF)], sidxb)
                pltpu.sync_copy(dst_hbm.at[pl.ds(rbase, _HALF)], didxb)

                for b in range(_NB):
                    start_gather(b, b)

                @pl.loop(0, nloops)
                def _(g):
                    for b in range(_NB):
                        j = g * _NB + b
                        wait_gather(j, b)
                        scatter(j, b)
                        start_gather(j + _NB, b)

                for b in range(_NB):
                    j = nloops * _NB + b
                    wait_gather(j, b)
                    scatter(j, b)

        @pl.when(c == 0)
        def _():
            pipe(s * _RT0, _RT0 // _HALF)

        @pl.when(c == 1)
        def _():
            pipe(_NS * _RT0 + s * _RT1, _RT1 // _HALF)

        plsc.subcore_barrier()
        _write_out(acc, out_hbm, c, s)

    return k(hp, src2d, dst2d, zeros_blk)


def _tc_matmul1(x, W1):
    def body(x_ref, w_ref, h_ref):
        h_ref[...] = jnp.dot(x_ref[...], w_ref[...],
                             preferred_element_type=jnp.float32)

    return pl.pallas_call(
        body,
        grid=(_N // _ROWBLK,),
        in_specs=[
            pl.BlockSpec((_ROWBLK, _D), lambda i: (i, 0)),
            pl.BlockSpec((_D, _D), lambda i: (0, 0)),
        ],
        out_specs=pl.BlockSpec((_ROWBLK, _D), lambda i: (i, 0)),
        out_shape=jax.ShapeDtypeStruct((_N, _D), jnp.float32),
    )(x, W1)


def _tc_scale(degp, h1):
    def body(degp_ref, h_ref, hp_ref, inv_ref):
        dp = degp_ref[0] + degp_ref[1]
        inv = lax.rsqrt(dp[:, 0:1] + 1.0)
        hp_ref[...] = h_ref[...] * inv
        inv_ref[...] = inv

    return pl.pallas_call(
        body,
        grid=(_N // _ROWBLK,),
        in_specs=[
            pl.BlockSpec((_NC, _ROWBLK, _D), lambda i: (0, i, 0)),
            pl.BlockSpec((_ROWBLK, _D), lambda i: (i, 0)),
        ],
        out_specs=[
            pl.BlockSpec((_ROWBLK, _D), lambda i: (i, 0)),
            pl.BlockSpec((_ROWBLK, 1), lambda i: (i, 0)),
        ],
        out_shape=[
            jax.ShapeDtypeStruct((_NH, _D), jnp.float32),
            jax.ShapeDtypeStruct((_N, 1), jnp.float32),
        ],
    )(degp, h1)


def _tc_combine_prep(part, hp, inv, b, W):
    def body(part_ref, hp_ref, inv_ref, b_ref, w_ref, out_ref):
        inv_v = inv_ref[...]
        t = (part_ref[0] + part_ref[1] + hp_ref[...]) * inv_v + b_ref[...]
        h = _gelu_exact(t)
        out_ref[...] = jnp.dot(h, w_ref[...],
                               preferred_element_type=jnp.float32) * inv_v

    return pl.pallas_call(
        body,
        grid=(_N // _ROWBLK,),
        in_specs=[
            pl.BlockSpec((_NC, _ROWBLK, _D), lambda i: (0, i, 0)),
            pl.BlockSpec((_ROWBLK, _D), lambda i: (i, 0)),
            pl.BlockSpec((_ROWBLK, 1), lambda i: (i, 0)),
            pl.BlockSpec((1, _D), lambda i: (0, 0)),
            pl.BlockSpec((_D, _D), lambda i: (0, 0)),
        ],
        out_specs=pl.BlockSpec((_ROWBLK, _D), lambda i: (i, 0)),
        out_shape=jax.ShapeDtypeStruct((_NH, _D), jnp.float32),
    )(part, hp, inv, b, W)


def _tc_final(part, hp, inv, b, batch3d, Wc, bc):
    steps = _N // _ROWBLK

    def body(part_ref, hp_ref, inv_ref, b_ref, bat_ref, wc_ref, bc_ref,
             out_ref, accs, accc):
        i = pl.program_id(0)

        @pl.when(i == 0)
        def _():
            accs[...] = jnp.zeros_like(accs)
            accc[...] = jnp.zeros_like(accc)

        inv_v = inv_ref[...]
        t = (part_ref[0] + part_ref[1] + hp_ref[...]) * inv_v + b_ref[...]
        h = _gelu_exact(t)
        oht = (bat_ref[0] ==
               lax.broadcasted_iota(jnp.int32, (_G, _ROWBLK), 0)
               ).astype(jnp.float32)
        accs[...] += jnp.dot(oht, h, preferred_element_type=jnp.float32)
        accc[...] += jnp.dot(oht, jnp.ones((_ROWBLK, 1), jnp.float32),
                             preferred_element_type=jnp.float32)

        @pl.when(i == steps - 1)
        def _():
            g = accs[...] / jnp.maximum(accc[...], 1.0)
            out_ref[...] = jnp.dot(g, wc_ref[...],
                                   preferred_element_type=jnp.float32) + bc_ref[...]

    return pl.pallas_call(
        body,
        grid=(steps,),
        in_specs=[
            pl.BlockSpec((_NC, _ROWBLK, _D), lambda i: (0, i, 0)),
            pl.BlockSpec((_ROWBLK, _D), lambda i: (i, 0)),
            pl.BlockSpec((_ROWBLK, 1), lambda i: (i, 0)),
            pl.BlockSpec((1, _D), lambda i: (0, 0)),
            pl.BlockSpec((1, 1, _ROWBLK), lambda i: (i, 0, 0)),
            pl.BlockSpec((_D, _C), lambda i: (0, 0)),
            pl.BlockSpec((1, _C), lambda i: (0, 0)),
        ],
        out_specs=pl.BlockSpec((_G, _C), lambda i: (0, 0)),
        out_shape=jax.ShapeDtypeStruct((_G, _C), jnp.float32),
        scratch_shapes=[
            pltpu.VMEM((_G, _D), jnp.float32),
            pltpu.VMEM((_G, 1), jnp.float32),
        ],
    )(part, hp, inv, b, batch3d, Wc, bc)


def kernel(x, edge_index, batch, W1, b1, W2, b2, Wc, bc):
    src2d = jnp.concatenate(
        [edge_index[0], jnp.zeros((_EPAD,), jnp.int32)]).reshape(_ERP, _EK)
    dst2d = jnp.concatenate(
        [edge_index[1], jnp.full((_EPAD,), _N, jnp.int32)]).reshape(_ERP, _EK)
    ones_blk = jnp.ones((_EK, _D), jnp.float32)
    zeros_blk = jnp.zeros((_ZR, _D), jnp.float32)

    degp = _sc_degree(dst2d, ones_blk, zeros_blk)
    h1 = _tc_matmul1(x, W1)
    hp1, inv = _tc_scale(degp, h1)
    part1 = _sc_scatter(hp1, src2d, dst2d, zeros_blk)
    hp2 = _tc_combine_prep(part1, hp1, inv, b1.reshape(1, _D), W2)
    part2 = _sc_scatter(hp2, src2d, dst2d, zeros_blk)
    return _tc_final(part2, hp2, inv, b2.reshape(1, _D),
                     batch.reshape(_N // _ROWBLK, 1, _ROWBLK), Wc,
                     bc.reshape(1, _C))


# speedup vs baseline: 1.0799x; 1.0799x over previous
import functools

import jax
import jax.numpy as jnp
from jax import lax
from jax.experimental import pallas as pl
from jax.experimental.pallas import tpu as pltpu
from jax.experimental.pallas import tpu_sc as plsc

_N = 10000
_E = 320000
_D = 128
_G = 64
_C = 10

_NC = 2
_NS = 16
_NW = _NC * _NS

_EK = 128
_RT = 80
_ERP = _NW * _RT
_EPAD = _ERP * _EK - _E
_NH = _N + 8
_NACC = _N + 8
_NB = 2
_GS = 4
_QR = _EK // _GS
_HALF = 40
_RT0 = 120
_RT1 = 40

_ZR = 632
_ZR_LAST = _N - 15 * _ZR

_ROWBLK = 1000


def _gelu_exact(x):
    return 0.5 * x * (1.0 + lax.erf(x * 0.7071067811865476))


def _zero_acc(zeros_hbm, acc, s):
    @pl.when(s < _NS - 1)
    def _():
        pltpu.sync_copy(zeros_hbm.at[pl.ds(0, _ZR)],
                        acc.at[pl.ds(s * _ZR, _ZR)])

    @pl.when(s == _NS - 1)
    def _():
        pltpu.sync_copy(zeros_hbm.at[pl.ds(0, _ZR_LAST)],
                        acc.at[pl.ds(15 * _ZR, _ZR_LAST)])


def _write_out(acc, out_hbm, c, s):
    @pl.when(s < _NS - 1)
    def _():
        pltpu.sync_copy(acc.at[pl.ds(s * _ZR, _ZR)],
                        out_hbm.at[c, pl.ds(s * _ZR, _ZR)])

    @pl.when(s == _NS - 1)
    def _():
        pltpu.sync_copy(acc.at[pl.ds(15 * _ZR, _ZR_LAST)],
                        out_hbm.at[c, pl.ds(15 * _ZR, _ZR_LAST)])


def _load_slab(src2d, buf, rbase):
    pltpu.sync_copy(src2d.at[pl.ds(rbase, _RT)], buf)


def _sc_degree(dst2d, ones_blk, zeros_blk):
    mesh = plsc.VectorSubcoreMesh(core_axis_name="c", subcore_axis_name="s")

    @functools.partial(
        pl.kernel,
        out_type=jax.ShapeDtypeStruct((_NC, _N, _D), jnp.float32),
        mesh=mesh,
        scratch_types=[
            pltpu.VMEM((_RT, _EK), jnp.int32),
            pltpu.VMEM((_EK, _D), jnp.float32),
            pltpu.VMEM_SHARED((_NACC, _D), jnp.float32),
        ] + [pltpu.SemaphoreType.DMA] * _NB,
    )
    def k(dst_hbm, ones_hbm, zeros_hbm, out_hbm, didxb, ones_v, acc, *sems):
        c = lax.axis_index("c")
        s = lax.axis_index("s")
        t = c * _NS + s
        rbase = t * _RT

        _zero_acc(zeros_hbm, acc, s)
        pltpu.sync_copy(ones_hbm, ones_v)
        _load_slab(dst_hbm, didxb, rbase)
        plsc.subcore_barrier()

        def start(j, b):
            pltpu.async_copy(ones_v, acc.at[didxb.at[j]], sems[b], add=True)

        def wait(j, b):
            pltpu.make_async_copy(ones_v, acc.at[didxb.at[j]], sems[b]).wait()

        for b in range(_NB):
            start(b, b)
        nloops = (_RT - _NB) // _NB

        @pl.loop(0, nloops)
        def _(g):
            for b in range(_NB):
                j = g * _NB + b
                wait(j, b)
                start(j + _NB, b)

        for b in range(_NB):
            j = nloops * _NB + b
            wait(j, b)

        plsc.subcore_barrier()
        _write_out(acc, out_hbm, c, s)

    return k(dst2d, ones_blk, zeros_blk)


def _sc_scatter(hp, src2d, dst2d, zeros_blk):
    mesh = plsc.VectorSubcoreMesh(core_axis_name="c", subcore_axis_name="s")

    @functools.partial(
        pl.kernel,
        out_type=jax.ShapeDtypeStruct((_NC, _N, _D), jnp.float32),
        mesh=mesh,
        scratch_types=[
            pltpu.VMEM((_HALF, _EK), jnp.int32),
            pltpu.VMEM((_HALF, _EK), jnp.int32),
            pltpu.VMEM((_NB, _EK, _D), jnp.float32),
            pltpu.VMEM_SHARED((_NACC, _D), jnp.float32),
        ] + [pltpu.SemaphoreType.DMA] * (_NB * _GS),
    )
    def k(hp_hbm, src_hbm, dst_hbm, zeros_hbm, out_hbm,
          sidxb, didxb, rows, acc, *sems):
        c = lax.axis_index("c")
        s = lax.axis_index("s")

        _zero_acc(zeros_hbm, acc, s)
        plsc.subcore_barrier()

        def start_gather(j, b):
            for q in range(_GS):
                pltpu.async_copy(
                    hp_hbm.at[sidxb.at[j, pl.ds(q * _QR, _QR)]],
                    rows.at[b, pl.ds(q * _QR, _QR)], sems[b * _GS + q])

        def wait_gather(j, b):
            for q in range(_GS):
                pltpu.make_async_copy(
                    hp_hbm.at[sidxb.at[j, pl.ds(q * _QR, _QR)]],
                    rows.at[b, pl.ds(q * _QR, _QR)], sems[b * _GS + q]).wait()

        def scatter(j, b):
            pltpu.sync_copy(rows.at[b], acc.at[didxb.at[j]], add=True)

        nloops = (_HALF - _NB) // _NB

        def pipe(base, nstages):
            for st in range(nstages):
                rbase = base + st * _HALF
                pltpu.sync_copy(src_hbm.at[pl.ds(rbase, _HALF)], sidxb)
                pltpu.sync_copy(dst_hbm.at[pl.ds(rbase, _HALF)], didxb)

                for b in range(_NB):
                    start_gather(b, b)

                @pl.loop(0, nloops)
                def _(g):
                    for b in range(_NB):
                        j = g * _NB + b
                        wait_gather(j, b)
                        scatter(j, b)
                        start_gather(j + _NB, b)

                for b in range(_NB):
                    j = nloops * _NB + b
                    wait_gather(j, b)
                    scatter(j, b)

        @pl.when(c == 0)
        def _():
            pipe(s * _RT0, _RT0 // _HALF)

        @pl.when(c == 1)
        def _():
            pipe(_NS * _RT0 + s * _RT1, _RT1 // _HALF)

        plsc.subcore_barrier()
        _write_out(acc, out_hbm, c, s)

    return k(hp, src2d, dst2d, zeros_blk)


def _tc_matmul1(x, W1):
    def body(x_ref, w_ref, h_ref):
        h_ref[...] = jnp.dot(x_ref[...], w_ref[...],
                             preferred_element_type=jnp.float32)

    return pl.pallas_call(
        body,
        grid=(_N // _ROWBLK,),
        in_specs=[
            pl.BlockSpec((_ROWBLK, _D), lambda i: (i, 0)),
            pl.BlockSpec((_D, _D), lambda i: (0, 0)),
        ],
        out_specs=pl.BlockSpec((_ROWBLK, _D), lambda i: (i, 0)),
        out_shape=jax.ShapeDtypeStruct((_N, _D), jnp.float32),
    )(x, W1)


def _tc_scale(degp, h1):
    def body(degp_ref, h_ref, hp_ref, inv_ref):
        dp = degp_ref[0] + degp_ref[1]
        inv = lax.rsqrt(dp[:, 0:1] + 1.0)
        hp_ref[...] = h_ref[...] * inv
        inv_ref[...] = inv

    return pl.pallas_call(
        body,
        grid=(_N // _ROWBLK,),
        in_specs=[
            pl.BlockSpec((_NC, _ROWBLK, _D), lambda i: (0, i, 0)),
            pl.BlockSpec((_ROWBLK, _D), lambda i: (i, 0)),
        ],
        out_specs=[
            pl.BlockSpec((_ROWBLK, _D), lambda i: (i, 0)),
            pl.BlockSpec((_ROWBLK, 1), lambda i: (i, 0)),
        ],
        out_shape=[
            jax.ShapeDtypeStruct((_NH, _D), jnp.float32),
            jax.ShapeDtypeStruct((_N, 1), jnp.float32),
        ],
    )(degp, h1)


def _tc_combine_prep(part, hp, inv, b, W):
    def body(part_ref, hp_ref, inv_ref, b_ref, w_ref, out_ref):
        inv_v = inv_ref[...]
        t = (part_ref[0] + part_ref[1] + hp_ref[...]) * inv_v + b_ref[...]
        h = _gelu_exact(t)
        out_ref[...] = jnp.dot(h, w_ref[...],
                               preferred_element_type=jnp.float32) * inv_v

    return pl.pallas_call(
        body,
        grid=(_N // _ROWBLK,),
        in_specs=[
            pl.BlockSpec((_NC, _ROWBLK, _D), lambda i: (0, i, 0)),
            pl.BlockSpec((_ROWBLK, _D), lambda i: (i, 0)),
            pl.BlockSpec((_ROWBLK, 1), lambda i: (i, 0)),
            pl.BlockSpec((1, _D), lambda i: (0, 0)),
            pl.BlockSpec((_D, _D), lambda i: (0, 0)),
        ],
        out_specs=pl.BlockSpec((_ROWBLK, _D), lambda i: (i, 0)),
        out_shape=jax.ShapeDtypeStruct((_NH, _D), jnp.float32),
    )(part, hp, inv, b, W)


def _tc_final(part, hp, inv, b, batch3d, Wc, bc):
    steps = _N // _ROWBLK

    def body(part_ref, hp_ref, inv_ref, b_ref, bat_ref, wc_ref, bc_ref,
             out_ref, accs, accc):
        i = pl.program_id(0)

        @pl.when(i == 0)
        def _():
            accs[...] = jnp.zeros_like(accs)
            accc[...] = jnp.zeros_like(accc)

        inv_v = inv_ref[...]
        t = (part_ref[0] + part_ref[1] + hp_ref[...]) * inv_v + b_ref[...]
        h = _gelu_exact(t)
        oht = (bat_ref[0] ==
               lax.broadcasted_iota(jnp.int32, (_G, _ROWBLK), 0)
               ).astype(jnp.float32)
        accs[...] += jnp.dot(oht, h, preferred_element_type=jnp.float32)
        accc[...] += jnp.dot(oht, jnp.ones((_ROWBLK, 1), jnp.float32),
                             preferred_element_type=jnp.float32)

        @pl.when(i == steps - 1)
        def _():
            g = accs[...] / jnp.maximum(accc[...], 1.0)
            out_ref[...] = jnp.dot(g, wc_ref[...],
                                   preferred_element_type=jnp.float32) + bc_ref[...]

    return pl.pallas_call(
        body,
        grid=(steps,),
        in_specs=[
            pl.BlockSpec((_NC, _ROWBLK, _D), lambda i: (0, i, 0)),
            pl.BlockSpec((_ROWBLK, _D), lambda i: (i, 0)),
            pl.BlockSpec((_ROWBLK, 1), lambda i: (i, 0)),
            pl.BlockSpec((1, _D), lambda i: (0, 0)),
            pl.BlockSpec((1, 1, _ROWBLK), lambda i: (i, 0, 0)),
            pl.BlockSpec((_D, _C), lambda i: (0, 0)),
            pl.BlockSpec((1, _C), lambda i: (0, 0)),
        ],
        out_specs=pl.BlockSpec((_G, _C), lambda i: (0, 0)),
        out_shape=jax.ShapeDtypeStruct((_G, _C), jnp.float32),
        scratch_shapes=[
            pltpu.VMEM((_G, _D), jnp.float32),
            pltpu.VMEM((_G, 1), jnp.float32),
        ],
    )(part, hp, inv, b, batch3d, Wc, bc)


def kernel(x, edge_index, batch, W1, b1, W2, b2, Wc, bc):
    src2d = jnp.concatenate(
        [edge_index[0], jnp.zeros((_EPAD,), jnp.int32)]).reshape(_ERP, _EK)
    dst2d = jnp.concatenate(
        [edge_index[1], jnp.full((_EPAD,), _N, jnp.int32)]).reshape(_ERP, _EK)
    ones_blk = jnp.ones((_EK, _D), jnp.float32)
    zeros_blk = jnp.zeros((_ZR, _D), jnp.float32)

    degp = _sc_degree(dst2d, ones_blk, zeros_blk)
    h1 = _tc_matmul1(x, W1)
    hp1, inv = _tc_scale(degp, h1)
    part1 = _sc_scatter(hp1, src2d, dst2d, zeros_blk)
    hp2 = _tc_combine_prep(part1, hp1, inv, b1.reshape(1, _D), W2)
    part2 = _sc_scatter(hp2, src2d, dst2d, zeros_blk)
    return _tc_final(part2, hp2, inv, b2.reshape(1, _D),
                     batch.reshape(_N // _ROWBLK, 1, _ROWBLK), Wc,
                     bc.reshape(1, _C))
